# baseline (speedup 1.0000x reference)
import jax
import jax.numpy as jnp
from jax.experimental import pallas as pl

D_MODEL = 768
D_FF = 3072
TOK_BLOCK = 512


def _ffn_block(x_ref, wi_ref, bi_ref, wo_ref, bo_ref, out_ref):
    x = x_ref[...].astype(jnp.bfloat16)
    h = jax.lax.dot_general(
        x, wi_ref[...],
        dimension_numbers=(((1,), (1,)), ((), ())),
        preferred_element_type=jnp.float32,
    )
    h = jnp.maximum(h + bi_ref[...], 0.0).astype(jnp.bfloat16)
    out = jax.lax.dot_general(
        h, wo_ref[...],
        dimension_numbers=(((1,), (1,)), ((), ())),
        preferred_element_type=jnp.float32,
    )
    out_ref[...] = out + bo_ref[...]


def kernel(hidden_states, wi, bi, wo, bo):
    b, s, d = hidden_states.shape
    n_tok = b * s
    x = hidden_states.reshape(n_tok, d)
    wi_bf = wi.astype(jnp.bfloat16)
    wo_bf = wo.astype(jnp.bfloat16)
    bi2 = bi.reshape(1, D_FF)
    bo2 = bo.reshape(1, D_MODEL)

    grid = (n_tok // TOK_BLOCK,)
    out = pl.pallas_call(
        _ffn_block,
        grid=grid,
        in_specs=[
            pl.BlockSpec((TOK_BLOCK, D_MODEL), lambda i: (i, 0)),
            pl.BlockSpec((D_FF, D_MODEL), lambda i: (0, 0)),
            pl.BlockSpec((1, D_FF), lambda i: (0, 0)),
            pl.BlockSpec((D_MODEL, D_FF), lambda i: (0, 0)),
            pl.BlockSpec((1, D_MODEL), lambda i: (0, 0)),
        ],
        out_specs=pl.BlockSpec((TOK_BLOCK, D_MODEL), lambda i: (i, 0)),
        out_shape=jax.ShapeDtypeStruct((n_tok, D_MODEL), jnp.float32),
    )(x, wi_bf, bi2, wo_bf, bo2)

    return (out.reshape(b, s, d), None)


# speedup vs baseline: 1.1179x; 1.1179x over previous
import jax
import jax.numpy as jnp
from jax.experimental import pallas as pl

D_MODEL = 768
D_FF = 3072
TOK_BLOCK = 1024


def _ffn_block(x_ref, wi_ref, bi_ref, wo_ref, bo_ref, out_ref):
    x = x_ref[...].astype(jnp.bfloat16)
    h = jax.lax.dot_general(
        x, wi_ref[...].astype(jnp.bfloat16),
        dimension_numbers=(((1,), (1,)), ((), ())),
        preferred_element_type=jnp.float32,
    )
    h = jnp.maximum(h + bi_ref[...], 0.0).astype(jnp.bfloat16)
    out = jax.lax.dot_general(
        h, wo_ref[...].astype(jnp.bfloat16),
        dimension_numbers=(((1,), (1,)), ((), ())),
        preferred_element_type=jnp.float32,
    )
    out_ref[...] = out + bo_ref[...]


def kernel(hidden_states, wi, bi, wo, bo):
    b, s, d = hidden_states.shape
    n_tok = b * s
    x = hidden_states.reshape(n_tok, d)
    bi2 = bi.reshape(1, D_FF)
    bo2 = bo.reshape(1, D_MODEL)

    grid = (n_tok // TOK_BLOCK,)
    out = pl.pallas_call(
        _ffn_block,
        grid=grid,
        in_specs=[
            pl.BlockSpec((TOK_BLOCK, D_MODEL), lambda i: (i, 0)),
            pl.BlockSpec((D_FF, D_MODEL), lambda i: (0, 0)),
            pl.BlockSpec((1, D_FF), lambda i: (0, 0)),
            pl.BlockSpec((D_MODEL, D_FF), lambda i: (0, 0)),
            pl.BlockSpec((1, D_MODEL), lambda i: (0, 0)),
        ],
        out_specs=pl.BlockSpec((TOK_BLOCK, D_MODEL), lambda i: (i, 0)),
        out_shape=jax.ShapeDtypeStruct((n_tok, D_MODEL), jnp.float32),
    )(x, wi, bi2, wo, bo2)

    return (out.reshape(b, s, d), None)
